# baseline (speedup 1.0000x reference)
import functools

import jax
import jax.numpy as jnp
from jax import lax
from jax.experimental import pallas as pl
from jax.experimental.pallas import tpu as pltpu
from jax.experimental.pallas import tpu_sc as plsc

_LANES = 16
_IGNORE_INDEX = -100


@functools.lru_cache(maxsize=None)
def _make_nll_kernel(B: int, C: int):
    num_subcores = 16
    b_per_w = B // num_subcores
    chunks = b_per_w // _LANES
    mesh = plsc.VectorSubcoreMesh(
        core_axis_name="c", subcore_axis_name="s", num_cores=1
    )

    @functools.partial(
        pl.kernel,
        out_type=jax.ShapeDtypeStruct((_LANES,), jnp.float32),
        mesh=mesh,
        compiler_params=pltpu.CompilerParams(needs_layout_passes=False),
        scratch_types=[
            pltpu.VMEM((b_per_w,), jnp.int32),
            pltpu.VMEM((8 * b_per_w,), jnp.int32),
            pltpu.VMEM_SHARED((B, 8, 128), jnp.float32),
            pltpu.VMEM((b_per_w, 1, 128), jnp.float32),
            pltpu.VMEM((2 * _LANES,), jnp.float32),
            pltpu.VMEM_SHARED((num_subcores * 2 * _LANES,), jnp.float32),
            pltpu.VMEM((num_subcores * 2 * _LANES,), jnp.float32),
            pltpu.VMEM((_LANES,), jnp.float32),
            pltpu.VMEM((_LANES,), jnp.float32),
            pltpu.SemaphoreType.DMA,
        ],
    )
    def nll_kernel(preds_hbm, tgt_hbm, out_hbm,
                   tgt_v, idx_v, tiles_sh, seg_v, part_v, shared, all_v,
                   res_v, bfly_v, sem):
        sid = lax.axis_index("s")
        base = sid * b_per_w

        pltpu.sync_copy(tgt_hbm.at[pl.ds(base, b_per_w)], tgt_v)

        lane = lax.iota(jnp.int32, _LANES)
        for j in range(chunks):
            sample = j * _LANES + lane
            plsc.store_scatter(idx_v, [sample * 8], (base + sample) >> 3)

        copies = []
        for j in range(chunks):
            t = tgt_v[pl.ds(j * _LANES, _LANES)]
            safe = jnp.minimum(jnp.maximum(t, 0), C - 1)
            c0vec = (safe >> 7) << 7
            for k in range(_LANES):
                s = j * _LANES + k
                r0 = pl.multiple_of(base + (s // 8) * 8, 8)
                c0 = pl.multiple_of(c0vec[k], 128)
                copies.append(pltpu.async_copy(
                    preds_hbm.at[pl.ds(r0, 8), pl.ds(c0, 128)],
                    tiles_sh.at[base + s],
                    sem,
                ))
        for cp in copies:
            cp.wait()

        for s in range(b_per_w):
            pltpu.sync_copy(
                tiles_sh.at[pl.ds(base + s, 1), pl.ds((base + s) & 7, 1)],
                seg_v.at[pl.ds(s, 1)],
            )

        zero = jnp.zeros((_LANES,), jnp.int32)
        acc = jnp.zeros((_LANES,), jnp.float32)
        cnt = jnp.zeros((_LANES,), jnp.float32)
        for j in range(chunks):
            t = tgt_v[pl.ds(j * _LANES, _LANES)]
            valid = t != _IGNORE_INDEX
            safe = jnp.minimum(jnp.maximum(t, 0), C - 1)
            sample = j * _LANES + lane
            v = plsc.load_gather(seg_v, [sample, zero, safe & 127])
            acc = acc + jnp.where(valid, -v, 0.0)
            cnt = cnt + jnp.where(valid, 1.0, 0.0)

        part_v[pl.ds(0, _LANES)] = acc
        part_v[pl.ds(_LANES, _LANES)] = cnt
        pltpu.sync_copy(part_v, shared.at[pl.ds(sid * 2 * _LANES, 2 * _LANES)])
        plsc.subcore_barrier()

        pltpu.sync_copy(shared, all_v)
        tot = jnp.zeros((_LANES,), jnp.float32)
        num = jnp.zeros((_LANES,), jnp.float32)
        for w in range(num_subcores):
            tot = tot + all_v[pl.ds(w * 2 * _LANES, _LANES)]
            num = num + all_v[pl.ds(w * 2 * _LANES + _LANES, _LANES)]

        def lane_sum(vec):
            for shift in (8, 4, 2, 1):
                bfly_v[...] = vec
                vec = vec + plsc.load_gather(bfly_v, [lane ^ shift])
            return vec

        s = lane_sum(tot)
        n = lane_sum(num)
        res_v[...] = s / jnp.maximum(n, 1.0)

        @pl.when(sid == 0)
        def _():
            pltpu.sync_copy(res_v, out_hbm)

    return nll_kernel


def kernel(predictions, targets):
    B, C = predictions.shape
    tgt = targets.astype(jnp.int32)
    out = _make_nll_kernel(B, C)(predictions, tgt)
    return out[0]
